# baseline (speedup 1.0000x reference)
import functools

import jax
import jax.numpy as jnp
from jax import lax
from jax.experimental import pallas as pl
from jax.experimental.pallas import tpu as pltpu
from jax.experimental.pallas import tpu_sc as plsc

_B, _N, _H, _M = 1024, 200, 128, 200
_EPS = 1e-12
_T = _B * _N
_NC, _NS, _L = 2, 16, 16
_NW = _NC * _NS
_TW = _T // _NW
_C = 128
_NCHUNK = _TW // _C
_G = _C // _L
_HV = _H // _L

def _bcast_lane(vec, idx):
    return lax.gather(
        vec, idx[:, None],
        dimension_numbers=lax.GatherDimensionNumbers(
            offset_dims=(), collapsed_slice_dims=(0,), start_index_map=(0,)),
        slice_sizes=(1,),
        mode=lax.GatherScatterMode.PROMISE_IN_BOUNDS)


def _sc_body(tbl_h, p_h, s_h, g_h, b_h, out_h,
             tbl_v, g_v, b_v, pidx_v, sidx_v, e_v, out_v):
    wid = lax.axis_index("s") * _NC + lax.axis_index("c")
    pltpu.sync_copy(tbl_h, tbl_v)
    pltpu.sync_copy(g_h, g_v)
    pltpu.sync_copy(b_h, b_v)
    base0 = wid * _TW
    lane = lax.iota(jnp.int32, _L)
    half = jnp.full((_L,), 0.5, jnp.float32)
    gs = [g_v[pl.ds(cv * _L, _L)] for cv in range(_HV)]
    bs = [b_v[pl.ds(cv * _L, _L)] for cv in range(_HV)]

    def chunk_body(k, carry):
        base = base0 + k * _C
        pltpu.sync_copy(p_h.at[pl.ds(base, _C)], pidx_v)
        pltpu.sync_copy(s_h.at[pl.ds(base, _C)], sidx_v)
        for g in range(_G):
            tok0 = g * _L
            pidx = pidx_v[pl.ds(tok0, _L)] * _H
            sidx = sidx_v[pl.ds(tok0, _L)] * _H
            nidx = ((lane + (base + tok0)) % _N) * _H

            def col_block(cb, accs):
                acc, acc2 = accs
                for cc in range(_L):
                    c = cb * _L + cc
                    csplat = jnp.zeros((_L,), jnp.int32) + c
                    vn = plsc.load_gather(tbl_v, [nidx + csplat])
                    vp = plsc.load_gather(tbl_v, [pidx + csplat])
                    vs = plsc.load_gather(tbl_v, [sidx + csplat])
                    e = vn + half * (vp + vs)
                    plsc.store_scatter(e_v, [lane * _H + csplat], e)
                    acc = acc + e
                    acc2 = acc2 + e * e
                return acc, acc2

            acc, acc2 = lax.fori_loop(
                0, _H // _L, col_block,
                (jnp.zeros((_L,), jnp.float32), jnp.zeros((_L,), jnp.float32)))
            mu = acc * (1.0 / _H)
            var = acc2 * (1.0 / _H) - mu * mu + _EPS
            yi = jnp.full((_L,), 0x5F3759DF, jnp.int32) - (
                plsc.bitcast(var, jnp.int32) >> 1)
            y = plsc.bitcast(yi, jnp.float32)
            for _ in range(3):
                y = y * (1.5 - 0.5 * var * y * y)

            def tok_body(t, _):
                tsplat = jnp.zeros((_L,), jnp.int32) + t
                mu_sp = _bcast_lane(mu, tsplat)
                inv_sp = _bcast_lane(y, tsplat)
                ebase = t * _H
                obase = (tok0 + t) * _H
                for cv in range(_HV):
                    ev = e_v[pl.ds(ebase + cv * _L, _L)]
                    res = (ev - mu_sp) * inv_sp * gs[cv] + bs[cv]
                    out_v[pl.ds(obase + cv * _L, _L)] = res
                return 0

            lax.fori_loop(0, _L, tok_body, 0)
        pltpu.sync_copy(out_v, out_h.at[pl.ds(base * _H, _C * _H)])
        return carry

    lax.fori_loop(0, _NCHUNK, chunk_body, 0)


def kernel(top_vecs, tok_struct_vec, sent_struct_vec, table, gamma, beta):
    del top_vecs, tok_struct_vec
    p_idx = sent_struct_vec[:, :, 0].reshape(_T).astype(jnp.int32)
    s_idx = sent_struct_vec[:, :, 1].reshape(_T).astype(jnp.int32)
    mesh = plsc.VectorSubcoreMesh(core_axis_name="c", subcore_axis_name="s")
    run = functools.partial(
        pl.kernel,
        mesh=mesh,
        compiler_params=pltpu.CompilerParams(needs_layout_passes=False),
        out_type=jax.ShapeDtypeStruct((_T * _H,), jnp.float32),
        scratch_types=[
            pltpu.VMEM((_M * _H,), jnp.float32),
            pltpu.VMEM((_H,), jnp.float32),
            pltpu.VMEM((_H,), jnp.float32),
            pltpu.VMEM((_C,), jnp.int32),
            pltpu.VMEM((_C,), jnp.int32),
            pltpu.VMEM((_L * _H,), jnp.float32),
            pltpu.VMEM((_C * _H,), jnp.float32),
        ],
    )(_sc_body)
    out = run(table.reshape(_M * _H), p_idx, s_idx, gamma, beta)
    return out.reshape(_B, _N, _H)


# speedup vs baseline: 1.8623x; 1.8623x over previous
import functools

import jax
import jax.numpy as jnp
from jax import lax
from jax.experimental import pallas as pl
from jax.experimental.pallas import tpu as pltpu
from jax.experimental.pallas import tpu_sc as plsc

_B, _N, _H, _M = 1024, 200, 128, 200
_EPS = 1e-12
_T = _B * _N
_NC, _NS, _L = 2, 16, 16
_NW = _NC * _NS
_TW = _T // _NW
_C = 128
_NCHUNK = _TW // _C
_G = _C // _L
_HV = _H // _L

def _bcast_lane(vec, idx):
    return lax.gather(
        vec, idx[:, None],
        dimension_numbers=lax.GatherDimensionNumbers(
            offset_dims=(), collapsed_slice_dims=(0,), start_index_map=(0,)),
        slice_sizes=(1,),
        mode=lax.GatherScatterMode.PROMISE_IN_BOUNDS)


def _sc_body(tbl_h, p_h, s_h, g_h, b_h, out_h,
             tbl_v, g_v, b_v, pidx_v, sidx_v, e_v, out_v):
    wid = lax.axis_index("s") * _NC + lax.axis_index("c")
    pltpu.sync_copy(tbl_h, tbl_v)
    pltpu.sync_copy(g_h, g_v)
    pltpu.sync_copy(b_h, b_v)
    base0 = wid * _TW
    lane = lax.iota(jnp.int32, _L)
    half = jnp.full((_L,), 0.5, jnp.float32)
    one = jnp.full((_L,), 1, jnp.int32)
    gs = [g_v[pl.ds(cv * _L, _L)] for cv in range(_HV)]
    bs = [b_v[pl.ds(cv * _L, _L)] for cv in range(_HV)]

    def chunk_body(k, carry):
        base = base0 + k * _C
        pltpu.sync_copy(p_h.at[pl.ds(base, _C)], pidx_v)
        pltpu.sync_copy(s_h.at[pl.ds(base, _C)], sidx_v)
        for g in range(_G):
            tok0 = g * _L
            pidx = pidx_v[pl.ds(tok0, _L)] * _H
            sidx = sidx_v[pl.ds(tok0, _L)] * _H
            nidx = ((lane + (base + tok0)) % _N) * _H
            zf = jnp.zeros((_L,), jnp.float32)

            @plsc.parallel_loop(
                0, _H, carry=(zf, zf, nidx, pidx, sidx, lane * _H),
                unroll=8)
            def _p1(c, cr):
                acc, acc2, ni, pi, si, ei = cr
                vn = plsc.load_gather(tbl_v, [ni])
                vp = plsc.load_gather(tbl_v, [pi])
                vs = plsc.load_gather(tbl_v, [si])
                e = vn + half * (vp + vs)
                plsc.store_scatter(e_v, [ei], e)
                return (acc + e, acc2 + e * e,
                        ni + one, pi + one, si + one, ei + one)

            acc, acc2 = _p1[0], _p1[1]
            mu = acc * (1.0 / _H)
            var = acc2 * (1.0 / _H) - mu * mu + _EPS
            yi = jnp.full((_L,), 0x5F3759DF, jnp.int32) - (
                plsc.bitcast(var, jnp.int32) >> 1)
            y = plsc.bitcast(yi, jnp.float32)
            for _ in range(3):
                y = y * (1.5 - 0.5 * var * y * y)

            @plsc.parallel_loop(0, _L, unroll=2)
            def _p2(t):
                tsplat = jnp.zeros((_L,), jnp.int32) + t
                mu_sp = _bcast_lane(mu, tsplat)
                inv_sp = _bcast_lane(y, tsplat)
                ebase = t * _H
                obase = (tok0 + t) * _H
                for cv in range(_HV):
                    ev = e_v[pl.ds(ebase + cv * _L, _L)]
                    res = (ev - mu_sp) * inv_sp * gs[cv] + bs[cv]
                    out_v[pl.ds(obase + cv * _L, _L)] = res
        pltpu.sync_copy(out_v, out_h.at[pl.ds(base * _H, _C * _H)])
        return carry

    lax.fori_loop(0, _NCHUNK, chunk_body, 0)


def kernel(top_vecs, tok_struct_vec, sent_struct_vec, table, gamma, beta):
    del top_vecs, tok_struct_vec
    p_idx = sent_struct_vec[:, :, 0].reshape(_T).astype(jnp.int32)
    s_idx = sent_struct_vec[:, :, 1].reshape(_T).astype(jnp.int32)
    mesh = plsc.VectorSubcoreMesh(core_axis_name="c", subcore_axis_name="s")
    run = functools.partial(
        pl.kernel,
        mesh=mesh,
        compiler_params=pltpu.CompilerParams(needs_layout_passes=False),
        out_type=jax.ShapeDtypeStruct((_T * _H,), jnp.float32),
        scratch_types=[
            pltpu.VMEM((_M * _H,), jnp.float32),
            pltpu.VMEM((_H,), jnp.float32),
            pltpu.VMEM((_H,), jnp.float32),
            pltpu.VMEM((_C,), jnp.int32),
            pltpu.VMEM((_C,), jnp.int32),
            pltpu.VMEM((_L * _H,), jnp.float32),
            pltpu.VMEM((_C * _H,), jnp.float32),
        ],
    )(_sc_body)
    out = run(table.reshape(_M * _H), p_idx, s_idx, gamma, beta)
    return out.reshape(_B, _N, _H)


# speedup vs baseline: 6.5698x; 3.5278x over previous
import functools

import jax
import jax.numpy as jnp
from jax import lax
from jax.experimental import pallas as pl
from jax.experimental.pallas import tpu as pltpu
from jax.experimental.pallas import tpu_sc as plsc

_B, _N, _H, _M = 1024, 200, 128, 200
_EPS = 1e-12
_T = _B * _N
_NC, _NS, _L = 2, 16, 16
_NW = _NC * _NS
_TW = _T // _NW
_C = 128
_NCHUNK = _TW // _C
_G = _C // _L
_HV = _H // _L
_HS = _H + 1
_ES = _L + 1

def _bcast_lane(vec, idx):
    return lax.gather(
        vec, idx[:, None],
        dimension_numbers=lax.GatherDimensionNumbers(
            offset_dims=(), collapsed_slice_dims=(0,), start_index_map=(0,)),
        slice_sizes=(1,),
        mode=lax.GatherScatterMode.PROMISE_IN_BOUNDS)


def _sc_body(tbl_h, p_h, s_h, g_h, b_h, out_h,
             tbl_v, g_v, b_v, pidx_v, sidx_v, e_v, out_v):
    wid = lax.axis_index("s") * _NC + lax.axis_index("c")
    pltpu.sync_copy(tbl_h, tbl_v)
    pltpu.sync_copy(g_h, g_v)
    pltpu.sync_copy(b_h, b_v)
    base0 = wid * _TW
    lane = lax.iota(jnp.int32, _L)
    half = jnp.full((_L,), 0.5, jnp.float32)
    one = jnp.full((_L,), 1, jnp.int32)
    es_step = jnp.full((_L,), _ES, jnp.int32)
    ebases = [(cv * _L + lane) * _ES for cv in range(_HV)]
    gs = [g_v[pl.ds(cv * _L, _L)] for cv in range(_HV)]
    bs = [b_v[pl.ds(cv * _L, _L)] for cv in range(_HV)]

    def chunk_body(k, carry):
        base = base0 + k * _C
        pltpu.sync_copy(p_h.at[pl.ds(base, _C)], pidx_v)
        pltpu.sync_copy(s_h.at[pl.ds(base, _C)], sidx_v)
        for g in range(_G):
            tok0 = g * _L
            pidx = pidx_v[pl.ds(tok0, _L)] * _HS
            sidx = sidx_v[pl.ds(tok0, _L)] * _HS
            nidx = ((lane + (base + tok0)) % _N) * _HS
            zf = jnp.zeros((_L,), jnp.float32)

            @plsc.parallel_loop(
                0, _H, carry=(zf, zf, nidx, pidx, sidx, lane),
                unroll=8)
            def _p1(c, cr):
                acc, acc2, ni, pi, si, ei = cr
                vn = plsc.load_gather(tbl_v, [ni])
                vp = plsc.load_gather(tbl_v, [pi])
                vs = plsc.load_gather(tbl_v, [si])
                e = vn + half * (vp + vs)
                plsc.store_scatter(e_v, [ei], e)
                return (acc + e, acc2 + e * e,
                        ni + one, pi + one, si + one, ei + es_step)

            acc, acc2 = _p1[0], _p1[1]
            mu = acc * (1.0 / _H)
            var = acc2 * (1.0 / _H) - mu * mu + _EPS
            yi = jnp.full((_L,), 0x5F3759DF, jnp.int32) - (
                plsc.bitcast(var, jnp.int32) >> 1)
            y = plsc.bitcast(yi, jnp.float32)
            for _ in range(3):
                y = y * (1.5 - 0.5 * var * y * y)

            @plsc.parallel_loop(0, _L, unroll=2)
            def _p2(t):
                tsplat = jnp.zeros((_L,), jnp.int32) + t
                mu_sp = _bcast_lane(mu, tsplat)
                inv_sp = _bcast_lane(y, tsplat)
                obase = (tok0 + t) * _H
                for cv in range(_HV):
                    ev = plsc.load_gather(e_v, [ebases[cv] + tsplat])
                    res = (ev - mu_sp) * inv_sp * gs[cv] + bs[cv]
                    out_v[pl.ds(obase + cv * _L, _L)] = res
        pltpu.sync_copy(out_v, out_h.at[pl.ds(base * _H, _C * _H)])
        return carry

    lax.fori_loop(0, _NCHUNK, chunk_body, 0)


def kernel(top_vecs, tok_struct_vec, sent_struct_vec, table, gamma, beta):
    del top_vecs, tok_struct_vec
    p_idx = sent_struct_vec[:, :, 0].reshape(_T).astype(jnp.int32)
    s_idx = sent_struct_vec[:, :, 1].reshape(_T).astype(jnp.int32)
    mesh = plsc.VectorSubcoreMesh(core_axis_name="c", subcore_axis_name="s")
    run = functools.partial(
        pl.kernel,
        mesh=mesh,
        compiler_params=pltpu.CompilerParams(needs_layout_passes=False),
        out_type=jax.ShapeDtypeStruct((_T * _H,), jnp.float32),
        scratch_types=[
            pltpu.VMEM((_M * _HS,), jnp.float32),
            pltpu.VMEM((_H,), jnp.float32),
            pltpu.VMEM((_H,), jnp.float32),
            pltpu.VMEM((_C,), jnp.int32),
            pltpu.VMEM((_C,), jnp.int32),
            pltpu.VMEM((_H * _ES,), jnp.float32),
            pltpu.VMEM((_C * _H,), jnp.float32),
        ],
    )(_sc_body)
    tbl_pad = jnp.pad(table, ((0, 0), (0, _HS - _H))).reshape(_M * _HS)
    out = run(tbl_pad, p_idx, s_idx, gamma, beta)
    return out.reshape(_B, _N, _H)
